# baseline (speedup 1.0000x reference)
import functools

import jax
import jax.numpy as jnp
from jax import lax
from jax.experimental import pallas as pl
from jax.experimental.pallas import tpu as pltpu
from jax.experimental.pallas import tpu_sc as plsc

N_BOXES = 5000
NMS_THRESH = 0.8
N_POST_NMS = 2000
MIN_SIZE = 0.0

_PAD = 5120
_ROWS = 40
_LANES = 128
_OUT_PAD = 2048
_NW = 320
_BI = 256
_BJ = 256
_RB = 512
_CHUNK = 128
_NCHUNK = _PAD // _CHUNK
_NREG = _PAD // 256


def _decode_body(vals_ref, ims_ref, coords_ref, scores_ref, kc_ref):
    sc = vals_ref[0]
    dx, dy, dw, dh = vals_ref[1], vals_ref[2], vals_ref[3], vals_ref[4]
    a0, a1, a2, a3 = vals_ref[5], vals_ref[6], vals_ref[7], vals_ref[8]

    widths = a2 - a0 + 1.0
    heights = a3 - a1 + 1.0
    ctr_x = a0 + 0.5 * widths
    ctr_y = a1 + 0.5 * heights
    pred_ctr_x = dx * widths + ctr_x
    pred_ctr_y = dy * heights + ctr_y
    pred_w = jnp.exp(dw) * widths
    pred_h = jnp.exp(dh) * heights

    im_h = ims_ref[:, 0:1]
    im_w = ims_ref[:, 1:2]
    x1 = jnp.clip(pred_ctr_x - 0.5 * pred_w, 0.0, im_w)
    y1 = jnp.clip(pred_ctr_y - 0.5 * pred_h, 0.0, im_h)
    x2 = jnp.clip(pred_ctr_x + 0.5 * pred_w, 0.0, im_w)
    y2 = jnp.clip(pred_ctr_y + 0.5 * pred_h, 0.0, im_h)

    ws = x2 - x1 + 1.0
    hs = y2 - y1 + 1.0
    row = lax.broadcasted_iota(jnp.int32, (_ROWS, _LANES), 0)
    col = lax.broadcasted_iota(jnp.int32, (_ROWS, _LANES), 1)
    flat = row * _LANES + col
    mask = jnp.logical_and(jnp.logical_and(ws >= MIN_SIZE, hs >= MIN_SIZE),
                           flat < N_BOXES)

    coords_ref[0] = x1
    coords_ref[1] = y1
    coords_ref[2] = x2
    coords_ref[3] = y2
    scores_ref[...] = jnp.where(mask, sc, jnp.float32(-jnp.inf))
    kc_ref[...] = jnp.broadcast_to(jnp.sum(jnp.where(mask, 1, 0)), (1, 1))


def _rank_body(scol_ref, srow_ref, out_ref):
    si = scol_ref[...]
    pid = pl.program_id(0)
    ii = pid * _RB + lax.broadcasted_iota(jnp.int32, (_RB, 1), 0)
    acc = jnp.zeros((_RB, 256), jnp.float32)
    for c in range(_PAD // 256):
        sj = srow_ref[0:1, c * 256:(c + 1) * 256]
        jj = c * 256 + lax.broadcasted_iota(jnp.int32, (1, 256), 1)
        before = jnp.logical_or(sj > si,
                                jnp.logical_and(sj == si, jj < ii))
        acc = acc + jnp.where(before, 1.0, 0.0)
    out_ref[...] = jnp.sum(acc, axis=1, keepdims=True).astype(jnp.int32)


def _scatter_sc(coords, ranks):
    info = plsc.get_sparse_core_info()
    mesh = plsc.VectorSubcoreMesh(core_axis_name="c", subcore_axis_name="s")

    @functools.partial(
        pl.kernel, mesh=mesh,
        compiler_params=pltpu.CompilerParams(needs_layout_passes=False, use_tc_tiling_on_sc=False),
        out_type=jax.ShapeDtypeStruct((4, _PAD), jnp.float32),
        scratch_types=[
            pltpu.VMEM((_PAD,), jnp.int32),
            pltpu.VMEM((_PAD,), jnp.float32),
            pltpu.VMEM((_PAD,), jnp.float32),
        ],
    )
    def k(coords_hbm, ranks_hbm, out_hbm, idx_v, val_v, dst_v):
        wid = lax.axis_index("s") * info.num_cores + lax.axis_index("c")
        for t in range(4):
            @pl.when(wid == t)
            def _(t=t):
                pltpu.sync_copy(ranks_hbm, idx_v)
                pltpu.sync_copy(coords_hbm.at[t], val_v)

                def lp(i, _):
                    iv = idx_v[pl.ds(i * 16, 16)]
                    vv = val_v[pl.ds(i * 16, 16)]
                    plsc.store_scatter(dst_v, [iv], vv)
                    return 0

                lax.fori_loop(0, _PAD // 16, lp, 0)
                pltpu.sync_copy(dst_v, out_hbm.at[t])

    return k(coords, ranks)


_NB = _PAD // _BI


def _tri(i, j):
    cond = j < _NB - i
    pi = jnp.where(cond, i, _NB - 1 - i)
    jc = jnp.where(cond, i + j, j - 1)
    return pi, jc


def _mask_body(cols_ref, rows_ref, m_ref, any_ref):
    pi, jc = _tri(pl.program_id(0), pl.program_id(1))

    if True:
        c4 = cols_ref[...]
        r4 = rows_ref[...]
        x1i, y1i = c4[:, 0:1], c4[:, 1:2]
        x2i, y2i = c4[:, 2:3], c4[:, 3:4]
        x1j, y1j = r4[0:1, :], r4[1:2, :]
        x2j, y2j = r4[2:3, :], r4[3:4, :]
        ii = pi * _BI + lax.broadcasted_iota(jnp.int32, (_BI, 1), 0)
        jj = jc * _BJ + lax.broadcasted_iota(jnp.int32, (1, _BJ), 1)
        area_i = (x2i - x1i) * (y2i - y1i)
        area_j = (x2j - x1j) * (y2j - y1j)
        xx1 = jnp.maximum(x1i, x1j)
        yy1 = jnp.maximum(y1i, y1j)
        xx2 = jnp.minimum(x2i, x2j)
        yy2 = jnp.minimum(y2i, y2j)
        inter = jnp.maximum(xx2 - xx1, 0.0) * jnp.maximum(yy2 - yy1, 0.0)
        union = area_i + area_j - inter
        iou = jnp.where(union > 0, inter / jnp.maximum(union, 1e-12), 0.0)
        kill = jnp.logical_and(iou > NMS_THRESH, jj > ii)
        kf = jnp.where(kill, 1.0, 0.0)
        rr = lax.broadcasted_iota(jnp.int32, (_BJ, _BJ // 16), 0)
        ww = lax.broadcasted_iota(jnp.int32, (_BJ, _BJ // 16), 1)
        w16 = jnp.where(rr // 16 == ww,
                        lax.shift_left(jnp.int32(1), rr % 16),
                        0).astype(jnp.float32)
        packed = lax.dot(kf, w16, precision=lax.Precision.HIGHEST)
        m_ref[...] = packed.astype(jnp.int32)[None]
        rs = jnp.sum(packed, axis=1, keepdims=True).astype(jnp.int32)
        any_ref[...] = jnp.where(jc == pi, rs, any_ref[...] + rs)


def _scan_sc(m3, anyrow, sorted_coords, kvec):
    info = plsc.get_sparse_core_info()
    mesh = plsc.VectorSubcoreMesh(core_axis_name="c", subcore_axis_name="s")

    @functools.partial(
        pl.kernel, mesh=mesh,
        compiler_params=pltpu.CompilerParams(needs_layout_passes=False, use_tc_tiling_on_sc=False),
        out_type=jax.ShapeDtypeStruct((_OUT_PAD * 4,), jnp.float32),
        scratch_types=[
            pltpu.VMEM((_NREG, _CHUNK, 16), jnp.int32),
            pltpu.VMEM((_NREG, _CHUNK, 16), jnp.int32),
            pltpu.VMEM((4, _PAD), jnp.float32),
            pltpu.VMEM((_PAD,), jnp.int32),
            pltpu.VMEM((_OUT_PAD,), jnp.int32),
            pltpu.VMEM((16,), jnp.int32),
            pltpu.VMEM((_NW,), jnp.int32),
            pltpu.VMEM((_OUT_PAD * 4,), jnp.float32),
            pltpu.SemaphoreType.DMA,
            pltpu.SemaphoreType.DMA,
        ],
    )
    def k(m_hbm, any_hbm, coords_hbm, k_hbm, out_hbm,
          mbufa, mbufb, coords_v, any_v, kept_v, kvec_v, sup_v, out_v,
          sema, semb):
        wid = lax.axis_index("s") * info.num_cores + lax.axis_index("c")

        @pl.when(wid == 0)
        def _():
            pltpu.sync_copy(k_hbm, kvec_v)
            pltpu.sync_copy(coords_hbm, coords_v)
            pltpu.sync_copy(any_hbm, any_v)
            iota16 = lax.iota(jnp.int32, 16)
            zi = jnp.zeros((16,), jnp.int32)
            zf = jnp.zeros((16,), jnp.float32)
            for t in range(_NREG):
                sup_v[pl.ds(t * 16, 16)] = zi
            kk = jnp.max(kvec_v[...])

            bufs = (mbufa, mbufb)
            sems = (sema, semb)
            handles = {}
            for c in range(2):
                handles[c] = pltpu.async_copy(
                    m_hbm.at[:, pl.ds(c * _CHUNK, _CHUNK), :],
                    bufs[c], sems[c])

            p = jnp.int32(0)
            cnt = jnp.int32(0)

            for c in range(_NCHUNK):
                buf = bufs[c % 2]
                handles[c].wait()
                t_c = c // 2
                pend = jnp.minimum(kk, jnp.int32((c + 1) * _CHUNK))

                def cond(st):
                    pp, cc = st
                    return jnp.logical_and(pp < pend, cc < N_POST_NMS)

                def body(st, c=c, t_c=t_c, buf=buf):
                    pp, cc = st
                    sup_tc = sup_v[pl.ds(t_c * 16, 16)]
                    w = pp >> 4
                    wvalv = lax.gather(
                        sup_tc,
                        jnp.broadcast_to(w & 15, (16,))[:, None],
                        lax.GatherDimensionNumbers(
                            offset_dims=(), collapsed_slice_dims=(0,),
                            start_index_map=(0,)),
                        (1,),
                        mode=lax.GatherScatterMode.PROMISE_IN_BOUNDS)
                    bits = (wvalv >> iota16) & 1
                    posv = (w << 4) + iota16
                    avail = jnp.logical_and(
                        jnp.logical_and(bits == 0, posv >= pp), posv < kk)
                    ar = plsc.load_gather(any_v, [posv])
                    blocked = jnp.logical_and(avail, ar != 0)
                    pr = plsc.cumsum(jnp.where(avail, 1, 0))
                    fb = jnp.min(jnp.where(blocked, iota16, 16))
                    batch = jnp.logical_and(avail, iota16 < fb)
                    allowed = jnp.logical_and(
                        batch, pr <= jnp.int32(N_POST_NMS) - cc)
                    nk = jnp.max(jnp.where(allowed, pr, 0))
                    plsc.store_scatter(kept_v, [cc - 1 + pr], posv,
                                       mask=allowed)
                    cc2 = cc + nk

                    def keep_one(op):
                        pp, cc = op
                        pos = (w << 4) + fb
                        lr = pos - c * _CHUNK
                        for t in range(t_c, _NREG):
                            sup_v[pl.ds(t * 16, 16)] = jnp.bitwise_or(
                                sup_v[pl.ds(t * 16, 16)], buf[t, lr, :])
                        plsc.store_scatter(
                            kept_v, [jnp.broadcast_to(cc, (16,))],
                            jnp.broadcast_to(pos, (16,)),
                            mask=iota16 == 0)
                        return pos + 1, cc + 1

                    def skip(op):
                        pp, cc = op
                        return (w + 1) << 4, cc

                    return lax.cond(
                        jnp.logical_and(fb < 16, cc2 < N_POST_NMS),
                        keep_one, skip, (pp, cc2))

                p, cnt = lax.while_loop(cond, body, (p, cnt))
                if c + 2 < _NCHUNK:
                    handles[c + 2] = pltpu.async_copy(
                        m_hbm.at[:, pl.ds((c + 2) * _CHUNK, _CHUNK), :],
                        buf, sems[c % 2])

            def glp(g, _):
                idx = kept_v[pl.ds(g * 16, 16)]
                valid = (g * 16 + iota16) < cnt
                base4 = (g * 16 + iota16) * 4
                for r in range(4):
                    vals = plsc.load_gather(
                        coords_v, [jnp.full((16,), r, jnp.int32), idx],
                        mask=valid)
                    plsc.store_scatter(out_v, [base4 + r], vals, mask=valid)
                return 0

            lax.fori_loop(0, (cnt + 15) >> 4, glp, 0)

            start = cnt * 4
            g0 = start >> 4
            plsc.store_scatter(out_v, [g0 * 16 + iota16], zf,
                               mask=(g0 * 16 + iota16) >= start)

            def zlp(g, _):
                out_v[pl.ds(g * 16, 16)] = zf
                return 0

            lax.fori_loop(g0 + 1, _OUT_PAD * 4 // 16, zlp, 0)
            pltpu.sync_copy(out_v, out_hbm)

    return k(m3, anyrow, sorted_coords, kvec)


def kernel(classification, regression, anchors, image_shape):
    sc = classification[0, :, classification.shape[-1] // 2]
    reg = jnp.reshape(regression[0], (-1, 4))
    anc = anchors[0]

    def p(v):
        return jnp.reshape(jnp.pad(v, (0, _PAD - N_BOXES)), (_ROWS, _LANES))

    vals = jnp.stack([
        p(sc),
        p(reg[:, 0]), p(reg[:, 1]), p(reg[:, 2]), p(reg[:, 3]),
        p(anc[:, 0]), p(anc[:, 1]), p(anc[:, 2]), p(anc[:, 3]),
    ])
    ims = jnp.reshape(image_shape.astype(jnp.float32), (1, 2))

    coords, scores, kc = pl.pallas_call(
        _decode_body,
        out_shape=(
            jax.ShapeDtypeStruct((4, _ROWS, _LANES), jnp.float32),
            jax.ShapeDtypeStruct((_ROWS, _LANES), jnp.float32),
            jax.ShapeDtypeStruct((1, 1), jnp.int32),
        ),
    )(vals, ims)

    scores_flat = jnp.reshape(scores, (_PAD,))
    ranks = pl.pallas_call(
        _rank_body,
        grid=(_PAD // _RB,),
        in_specs=[
            pl.BlockSpec((_RB, 1), lambda i: (i, 0)),
            pl.BlockSpec((1, _PAD), lambda i: (0, 0)),
        ],
        out_specs=pl.BlockSpec((_RB, 1), lambda i: (i, 0)),
        out_shape=jax.ShapeDtypeStruct((_PAD, 1), jnp.int32),
    )(scores_flat[:, None], scores_flat[None, :])

    sorted_coords = _scatter_sc(jnp.reshape(coords, (4, _PAD)),
                                jnp.reshape(ranks, (_PAD,)))

    m = pl.pallas_call(
        _mask_body,
        grid=(_NB // 2, _NB + 1),
        in_specs=[
            pl.BlockSpec((_BI, 4), lambda i, j: (_tri(i, j)[0], 0)),
            pl.BlockSpec((4, _BJ), lambda i, j: (0, _tri(i, j)[1])),
        ],
        out_specs=(
            pl.BlockSpec((1, _BI, _BJ // 16),
                         lambda i, j: (_tri(i, j)[1], _tri(i, j)[0], 0)),
            pl.BlockSpec((_BI, 1), lambda i, j: (_tri(i, j)[0], 0)),
        ),
        out_shape=(
            jax.ShapeDtypeStruct((_NW // 16, _PAD, 16), jnp.int32),
            jax.ShapeDtypeStruct((_PAD, 1), jnp.int32),
        ),
    )(jnp.transpose(sorted_coords), sorted_coords)

    kvec = jnp.full((16,), 1, jnp.int32) * kc[0, 0]
    out_flat = _scan_sc(m[0], jnp.reshape(m[1], (_PAD,)), sorted_coords, kvec)
    return jnp.reshape(out_flat, (_OUT_PAD, 4))[None, :N_POST_NMS, :]


# speedup vs baseline: 1.0643x; 1.0643x over previous
import functools

import jax
import jax.numpy as jnp
from jax import lax
from jax.experimental import pallas as pl
from jax.experimental.pallas import tpu as pltpu
from jax.experimental.pallas import tpu_sc as plsc

N_BOXES = 5000
NMS_THRESH = 0.8
N_POST_NMS = 2000
MIN_SIZE = 0.0

_PAD = 5120
_ROWS = 40
_LANES = 128
_OUT_PAD = 2048
_NW = 320
_BI = 256
_BJ = 256
_RB = 512
_CHUNK = 128
_NCHUNK = _PAD // _CHUNK
_NREG = _PAD // 256


def _decode_body(vals_ref, ims_ref, coords_ref, scores_ref, kc_ref):
    sc = vals_ref[0]
    dx, dy, dw, dh = vals_ref[1], vals_ref[2], vals_ref[3], vals_ref[4]
    a0, a1, a2, a3 = vals_ref[5], vals_ref[6], vals_ref[7], vals_ref[8]

    widths = a2 - a0 + 1.0
    heights = a3 - a1 + 1.0
    ctr_x = a0 + 0.5 * widths
    ctr_y = a1 + 0.5 * heights
    pred_ctr_x = dx * widths + ctr_x
    pred_ctr_y = dy * heights + ctr_y
    pred_w = jnp.exp(dw) * widths
    pred_h = jnp.exp(dh) * heights

    im_h = ims_ref[:, 0:1]
    im_w = ims_ref[:, 1:2]
    x1 = jnp.clip(pred_ctr_x - 0.5 * pred_w, 0.0, im_w)
    y1 = jnp.clip(pred_ctr_y - 0.5 * pred_h, 0.0, im_h)
    x2 = jnp.clip(pred_ctr_x + 0.5 * pred_w, 0.0, im_w)
    y2 = jnp.clip(pred_ctr_y + 0.5 * pred_h, 0.0, im_h)

    ws = x2 - x1 + 1.0
    hs = y2 - y1 + 1.0
    row = lax.broadcasted_iota(jnp.int32, (_ROWS, _LANES), 0)
    col = lax.broadcasted_iota(jnp.int32, (_ROWS, _LANES), 1)
    flat = row * _LANES + col
    mask = jnp.logical_and(jnp.logical_and(ws >= MIN_SIZE, hs >= MIN_SIZE),
                           flat < N_BOXES)

    coords_ref[0] = x1
    coords_ref[1] = y1
    coords_ref[2] = x2
    coords_ref[3] = y2
    scores_ref[...] = jnp.where(mask, sc, jnp.float32(-jnp.inf))
    kc_ref[...] = jnp.broadcast_to(jnp.sum(jnp.where(mask, 1, 0)), (1, 1))


def _rank_body(scol_ref, srow_ref, out_ref):
    si = scol_ref[...]
    pid = pl.program_id(0)
    ii = pid * _RB + lax.broadcasted_iota(jnp.int32, (_RB, 1), 0)
    acc = jnp.zeros((_RB, 256), jnp.float32)
    for c in range(_PAD // 256):
        sj = srow_ref[0:1, c * 256:(c + 1) * 256]
        jj = c * 256 + lax.broadcasted_iota(jnp.int32, (1, 256), 1)
        before = jnp.logical_or(sj > si,
                                jnp.logical_and(sj == si, jj < ii))
        acc = acc + jnp.where(before, 1.0, 0.0)
    out_ref[...] = jnp.sum(acc, axis=1, keepdims=True).astype(jnp.int32)


def _scatter_sc(coords, ranks):
    info = plsc.get_sparse_core_info()
    mesh = plsc.VectorSubcoreMesh(core_axis_name="c", subcore_axis_name="s")

    @functools.partial(
        pl.kernel, mesh=mesh,
        compiler_params=pltpu.CompilerParams(needs_layout_passes=False, use_tc_tiling_on_sc=False),
        out_type=jax.ShapeDtypeStruct((4, _PAD), jnp.float32),
        scratch_types=[
            pltpu.VMEM((_PAD,), jnp.int32),
            pltpu.VMEM((_PAD,), jnp.float32),
            pltpu.VMEM((_PAD,), jnp.float32),
        ],
    )
    def k(coords_hbm, ranks_hbm, out_hbm, idx_v, val_v, dst_v):
        wid = lax.axis_index("s") * info.num_cores + lax.axis_index("c")
        for t in range(4):
            @pl.when(wid == t)
            def _(t=t):
                pltpu.sync_copy(ranks_hbm, idx_v)
                pltpu.sync_copy(coords_hbm.at[t], val_v)

                def lp(i, _):
                    iv = idx_v[pl.ds(i * 16, 16)]
                    vv = val_v[pl.ds(i * 16, 16)]
                    plsc.store_scatter(dst_v, [iv], vv)
                    return 0

                lax.fori_loop(0, _PAD // 16, lp, 0)
                pltpu.sync_copy(dst_v, out_hbm.at[t])

    return k(coords, ranks)


_NB = _PAD // _BI


def _tri(i, j):
    cond = j < _NB - i
    pi = jnp.where(cond, i, _NB - 1 - i)
    jc = jnp.where(cond, i + j, j - 1)
    return pi, jc


def _mask_body(cols_ref, rows_ref, m_ref, any_ref):
    pi, jc = _tri(pl.program_id(0), pl.program_id(1))

    if True:
        c4 = cols_ref[...]
        r4 = rows_ref[...]
        x1i, y1i = c4[:, 0:1], c4[:, 1:2]
        x2i, y2i = c4[:, 2:3], c4[:, 3:4]
        x1j, y1j = r4[0:1, :], r4[1:2, :]
        x2j, y2j = r4[2:3, :], r4[3:4, :]
        ii = pi * _BI + lax.broadcasted_iota(jnp.int32, (_BI, 1), 0)
        jj = jc * _BJ + lax.broadcasted_iota(jnp.int32, (1, _BJ), 1)
        area_i = (x2i - x1i) * (y2i - y1i)
        area_j = (x2j - x1j) * (y2j - y1j)
        xx1 = jnp.maximum(x1i, x1j)
        yy1 = jnp.maximum(y1i, y1j)
        xx2 = jnp.minimum(x2i, x2j)
        yy2 = jnp.minimum(y2i, y2j)
        inter = jnp.maximum(xx2 - xx1, 0.0) * jnp.maximum(yy2 - yy1, 0.0)
        union = area_i + area_j - inter
        iou = jnp.where(union > 0, inter / jnp.maximum(union, 1e-12), 0.0)
        kill = jnp.logical_and(iou > NMS_THRESH, jj > ii)
        kf = jnp.where(kill, 1.0, 0.0)
        rr = lax.broadcasted_iota(jnp.int32, (_BJ, _BJ // 16), 0)
        ww = lax.broadcasted_iota(jnp.int32, (_BJ, _BJ // 16), 1)
        w16 = jnp.where(rr // 16 == ww,
                        lax.shift_left(jnp.int32(1), rr % 16),
                        0).astype(jnp.float32)
        packed = lax.dot(kf, w16, precision=lax.Precision.HIGHEST)
        m_ref[...] = jnp.reshape(packed.astype(jnp.int32), (2, 1, 128, 16))
        rs = jnp.sum(packed, axis=1, keepdims=True).astype(jnp.int32)
        any_ref[...] = jnp.where(jc == pi, rs, any_ref[...] + rs)


def _scan_sc(m3, anyrow, sorted_coords, kvec):
    info = plsc.get_sparse_core_info()
    mesh = plsc.VectorSubcoreMesh(core_axis_name="c", subcore_axis_name="s")

    @functools.partial(
        pl.kernel, mesh=mesh,
        compiler_params=pltpu.CompilerParams(needs_layout_passes=False, use_tc_tiling_on_sc=False),
        out_type=jax.ShapeDtypeStruct((_OUT_PAD * 4,), jnp.float32),
        scratch_types=[
            pltpu.VMEM((_NREG, _CHUNK, 16), jnp.int32),
            pltpu.VMEM((_NREG, _CHUNK, 16), jnp.int32),
            pltpu.VMEM((4, _PAD), jnp.float32),
            pltpu.VMEM((_PAD,), jnp.int32),
            pltpu.VMEM((_OUT_PAD,), jnp.int32),
            pltpu.VMEM((16,), jnp.int32),
            pltpu.VMEM((_NW,), jnp.int32),
            pltpu.VMEM((_OUT_PAD * 4,), jnp.float32),
            pltpu.SemaphoreType.DMA,
            pltpu.SemaphoreType.DMA,
        ],
    )
    def k(m_hbm, any_hbm, coords_hbm, k_hbm, out_hbm,
          mbufa, mbufb, coords_v, any_v, kept_v, kvec_v, sup_v, out_v,
          sema, semb):
        wid = lax.axis_index("s") * info.num_cores + lax.axis_index("c")

        @pl.when(wid == 0)
        def _():
            pltpu.sync_copy(k_hbm, kvec_v)
            pltpu.sync_copy(coords_hbm, coords_v)
            pltpu.sync_copy(any_hbm, any_v)
            iota16 = lax.iota(jnp.int32, 16)
            zi = jnp.zeros((16,), jnp.int32)
            zf = jnp.zeros((16,), jnp.float32)
            for t in range(_NREG):
                sup_v[pl.ds(t * 16, 16)] = zi
            kk = jnp.max(kvec_v[...])

            bufs = (mbufa, mbufb)
            sems = (sema, semb)

            def start_dma(c):
                t0 = c // 2
                return pltpu.async_copy(
                    m_hbm.at[c, pl.ds(t0, _NREG - t0)],
                    bufs[c % 2].at[pl.ds(t0, _NREG - t0)],
                    sems[c % 2])

            handles = {}
            for c in range(2):
                handles[c] = start_dma(c)

            p = jnp.int32(0)
            cnt = jnp.int32(0)

            for c in range(_NCHUNK):
                buf = bufs[c % 2]
                handles[c].wait()
                t_c = c // 2
                pend = jnp.minimum(kk, jnp.int32((c + 1) * _CHUNK))

                def cond(st):
                    pp, cc = st
                    return jnp.logical_and(pp < pend, cc < N_POST_NMS)

                def body(st, c=c, t_c=t_c, buf=buf):
                    pp, cc = st
                    sup_tc = sup_v[pl.ds(t_c * 16, 16)]
                    w = pp >> 4
                    wvalv = lax.gather(
                        sup_tc,
                        jnp.broadcast_to(w & 15, (16,))[:, None],
                        lax.GatherDimensionNumbers(
                            offset_dims=(), collapsed_slice_dims=(0,),
                            start_index_map=(0,)),
                        (1,),
                        mode=lax.GatherScatterMode.PROMISE_IN_BOUNDS)
                    bits = (wvalv >> iota16) & 1
                    posv = (w << 4) + iota16
                    avail = jnp.logical_and(
                        jnp.logical_and(bits == 0, posv >= pp), posv < kk)
                    ar = plsc.load_gather(any_v, [posv])
                    blocked = jnp.logical_and(avail, ar != 0)
                    pr = plsc.cumsum(jnp.where(avail, 1, 0))
                    fb = jnp.min(jnp.where(blocked, iota16, 16))
                    batch = jnp.logical_and(avail, iota16 < fb)
                    allowed = jnp.logical_and(
                        batch, pr <= jnp.int32(N_POST_NMS) - cc)
                    nk = jnp.max(jnp.where(allowed, pr, 0))
                    plsc.store_scatter(kept_v, [cc - 1 + pr], posv,
                                       mask=allowed)
                    cc2 = cc + nk

                    def keep_one(op):
                        pp, cc = op
                        pos = (w << 4) + fb
                        lr = pos - c * _CHUNK
                        for t in range(t_c, _NREG):
                            sup_v[pl.ds(t * 16, 16)] = jnp.bitwise_or(
                                sup_v[pl.ds(t * 16, 16)], buf[t, lr, :])
                        plsc.store_scatter(
                            kept_v, [jnp.broadcast_to(cc, (16,))],
                            jnp.broadcast_to(pos, (16,)),
                            mask=iota16 == 0)
                        return pos + 1, cc + 1

                    def skip(op):
                        pp, cc = op
                        return (w + 1) << 4, cc

                    return lax.cond(
                        jnp.logical_and(fb < 16, cc2 < N_POST_NMS),
                        keep_one, skip, (pp, cc2))

                p, cnt = lax.while_loop(cond, body, (p, cnt))
                if c + 2 < _NCHUNK:
                    handles[c + 2] = start_dma(c + 2)

            def glp(g, _):
                idx = kept_v[pl.ds(g * 16, 16)]
                valid = (g * 16 + iota16) < cnt
                base4 = (g * 16 + iota16) * 4
                for r in range(4):
                    vals = plsc.load_gather(
                        coords_v, [jnp.full((16,), r, jnp.int32), idx],
                        mask=valid)
                    plsc.store_scatter(out_v, [base4 + r], vals, mask=valid)
                return 0

            lax.fori_loop(0, (cnt + 15) >> 4, glp, 0)

            start = cnt * 4
            g0 = start >> 4
            plsc.store_scatter(out_v, [g0 * 16 + iota16], zf,
                               mask=(g0 * 16 + iota16) >= start)

            def zlp(g, _):
                out_v[pl.ds(g * 16, 16)] = zf
                return 0

            lax.fori_loop(g0 + 1, _OUT_PAD * 4 // 16, zlp, 0)
            pltpu.sync_copy(out_v, out_hbm)

    return k(m3, anyrow, sorted_coords, kvec)


def kernel(classification, regression, anchors, image_shape):
    sc = classification[0, :, classification.shape[-1] // 2]
    reg = jnp.reshape(regression[0], (-1, 4))
    anc = anchors[0]

    def p(v):
        return jnp.reshape(jnp.pad(v, (0, _PAD - N_BOXES)), (_ROWS, _LANES))

    vals = jnp.stack([
        p(sc),
        p(reg[:, 0]), p(reg[:, 1]), p(reg[:, 2]), p(reg[:, 3]),
        p(anc[:, 0]), p(anc[:, 1]), p(anc[:, 2]), p(anc[:, 3]),
    ])
    ims = jnp.reshape(image_shape.astype(jnp.float32), (1, 2))

    coords, scores, kc = pl.pallas_call(
        _decode_body,
        out_shape=(
            jax.ShapeDtypeStruct((4, _ROWS, _LANES), jnp.float32),
            jax.ShapeDtypeStruct((_ROWS, _LANES), jnp.float32),
            jax.ShapeDtypeStruct((1, 1), jnp.int32),
        ),
    )(vals, ims)

    scores_flat = jnp.reshape(scores, (_PAD,))
    ranks = pl.pallas_call(
        _rank_body,
        grid=(_PAD // _RB,),
        in_specs=[
            pl.BlockSpec((_RB, 1), lambda i: (i, 0)),
            pl.BlockSpec((1, _PAD), lambda i: (0, 0)),
        ],
        out_specs=pl.BlockSpec((_RB, 1), lambda i: (i, 0)),
        out_shape=jax.ShapeDtypeStruct((_PAD, 1), jnp.int32),
    )(scores_flat[:, None], scores_flat[None, :])

    sorted_coords = _scatter_sc(jnp.reshape(coords, (4, _PAD)),
                                jnp.reshape(ranks, (_PAD,)))

    m = pl.pallas_call(
        _mask_body,
        grid=(_NB // 2, _NB + 1),
        in_specs=[
            pl.BlockSpec((_BI, 4), lambda i, j: (_tri(i, j)[0], 0)),
            pl.BlockSpec((4, _BJ), lambda i, j: (0, _tri(i, j)[1])),
        ],
        out_specs=(
            pl.BlockSpec((2, 1, _CHUNK, 16),
                         lambda i, j: (_tri(i, j)[0], _tri(i, j)[1], 0, 0)),
            pl.BlockSpec((_BI, 1), lambda i, j: (_tri(i, j)[0], 0)),
        ),
        out_shape=(
            jax.ShapeDtypeStruct((_NCHUNK, _NREG, _CHUNK, 16), jnp.int32),
            jax.ShapeDtypeStruct((_PAD, 1), jnp.int32),
        ),
    )(jnp.transpose(sorted_coords), sorted_coords)

    kvec = jnp.full((16,), 1, jnp.int32) * kc[0, 0]
    out_flat = _scan_sc(m[0], jnp.reshape(m[1], (_PAD,)), sorted_coords, kvec)
    return jnp.reshape(out_flat, (_OUT_PAD, 4))[None, :N_POST_NMS, :]


# speedup vs baseline: 1.0821x; 1.0167x over previous
import functools

import jax
import jax.numpy as jnp
from jax import lax
from jax.experimental import pallas as pl
from jax.experimental.pallas import tpu as pltpu
from jax.experimental.pallas import tpu_sc as plsc

N_BOXES = 5000
NMS_THRESH = 0.8
N_POST_NMS = 2000
MIN_SIZE = 0.0

_PAD = 5120
_ROWS = 40
_LANES = 128
_OUT_PAD = 2048
_NW = 320
_BI = 256
_BJ = 256
_RB = 512
_CHUNK = 128
_NCHUNK = _PAD // _CHUNK
_NREG = _PAD // 256


def _decode_body(vals_ref, ims_ref, coords_ref, scores_ref, kc_ref):
    sc = vals_ref[0]
    dx, dy, dw, dh = vals_ref[1], vals_ref[2], vals_ref[3], vals_ref[4]
    a0, a1, a2, a3 = vals_ref[5], vals_ref[6], vals_ref[7], vals_ref[8]

    widths = a2 - a0 + 1.0
    heights = a3 - a1 + 1.0
    ctr_x = a0 + 0.5 * widths
    ctr_y = a1 + 0.5 * heights
    pred_ctr_x = dx * widths + ctr_x
    pred_ctr_y = dy * heights + ctr_y
    pred_w = jnp.exp(dw) * widths
    pred_h = jnp.exp(dh) * heights

    im_h = ims_ref[:, 0:1]
    im_w = ims_ref[:, 1:2]
    x1 = jnp.clip(pred_ctr_x - 0.5 * pred_w, 0.0, im_w)
    y1 = jnp.clip(pred_ctr_y - 0.5 * pred_h, 0.0, im_h)
    x2 = jnp.clip(pred_ctr_x + 0.5 * pred_w, 0.0, im_w)
    y2 = jnp.clip(pred_ctr_y + 0.5 * pred_h, 0.0, im_h)

    ws = x2 - x1 + 1.0
    hs = y2 - y1 + 1.0
    row = lax.broadcasted_iota(jnp.int32, (_ROWS, _LANES), 0)
    col = lax.broadcasted_iota(jnp.int32, (_ROWS, _LANES), 1)
    flat = row * _LANES + col
    mask = jnp.logical_and(jnp.logical_and(ws >= MIN_SIZE, hs >= MIN_SIZE),
                           flat < N_BOXES)

    coords_ref[0] = x1
    coords_ref[1] = y1
    coords_ref[2] = x2
    coords_ref[3] = y2
    scores_ref[...] = jnp.where(mask, sc, jnp.float32(-jnp.inf))
    kc_ref[...] = jnp.broadcast_to(jnp.sum(jnp.where(mask, 1, 0)), (1, 1))


def _rank_body(scol_ref, srow_ref, out_ref):
    si = jnp.transpose(scol_ref[...])
    pid = pl.program_id(0)
    ii = pid * _RB + lax.broadcasted_iota(jnp.int32, (_RB, 1), 0)
    acc = jnp.zeros((_RB, 256), jnp.float32)
    for c in range(_PAD // 256):
        sj = srow_ref[0:1, c * 256:(c + 1) * 256]
        jj = c * 256 + lax.broadcasted_iota(jnp.int32, (1, 256), 1)
        before = jnp.logical_or(sj > si,
                                jnp.logical_and(sj == si, jj < ii))
        acc = acc + jnp.where(before, 1.0, 0.0)
    out_ref[...] = jnp.transpose(
        jnp.sum(acc, axis=1, keepdims=True).astype(jnp.int32))


def _scatter_sc(coords, ranks):
    info = plsc.get_sparse_core_info()
    mesh = plsc.VectorSubcoreMesh(core_axis_name="c", subcore_axis_name="s")

    @functools.partial(
        pl.kernel, mesh=mesh,
        compiler_params=pltpu.CompilerParams(needs_layout_passes=False, use_tc_tiling_on_sc=False),
        out_type=jax.ShapeDtypeStruct((4, _PAD), jnp.float32),
        scratch_types=[
            pltpu.VMEM((_PAD,), jnp.int32),
            pltpu.VMEM((_PAD,), jnp.float32),
            pltpu.VMEM((_PAD,), jnp.float32),
        ],
    )
    def k(coords_hbm, ranks_hbm, out_hbm, idx_v, val_v, dst_v):
        wid = lax.axis_index("s") * info.num_cores + lax.axis_index("c")
        for t in range(4):
            @pl.when(wid == t)
            def _(t=t):
                pltpu.sync_copy(ranks_hbm, idx_v)
                pltpu.sync_copy(coords_hbm.at[t], val_v)

                def lp(i, _):
                    iv = idx_v[pl.ds(i * 16, 16)]
                    vv = val_v[pl.ds(i * 16, 16)]
                    plsc.store_scatter(dst_v, [iv], vv)
                    return 0

                lax.fori_loop(0, _PAD // 16, lp, 0)
                pltpu.sync_copy(dst_v, out_hbm.at[t])

    return k(coords, ranks)


_NB = _PAD // _BI


def _tri(i, j):
    cond = j < _NB - i
    pi = jnp.where(cond, i, _NB - 1 - i)
    jc = jnp.where(cond, i + j, j - 1)
    return pi, jc


def _mask_body(cols_ref, rows_ref, m_ref, any_ref):
    pi, jc = _tri(pl.program_id(0), pl.program_id(1))

    if True:
        c4 = cols_ref[...]
        r4 = rows_ref[...]
        x1i, y1i = c4[:, 0:1], c4[:, 1:2]
        x2i, y2i = c4[:, 2:3], c4[:, 3:4]
        x1j, y1j = r4[0:1, :], r4[1:2, :]
        x2j, y2j = r4[2:3, :], r4[3:4, :]
        ii = pi * _BI + lax.broadcasted_iota(jnp.int32, (_BI, 1), 0)
        jj = jc * _BJ + lax.broadcasted_iota(jnp.int32, (1, _BJ), 1)
        area_i = (x2i - x1i) * (y2i - y1i)
        area_j = (x2j - x1j) * (y2j - y1j)
        xx1 = jnp.maximum(x1i, x1j)
        yy1 = jnp.maximum(y1i, y1j)
        xx2 = jnp.minimum(x2i, x2j)
        yy2 = jnp.minimum(y2i, y2j)
        inter = jnp.maximum(xx2 - xx1, 0.0) * jnp.maximum(yy2 - yy1, 0.0)
        union = area_i + area_j - inter
        iou = jnp.where(union > 0, inter / jnp.maximum(union, 1e-12), 0.0)
        kill = jnp.logical_and(iou > NMS_THRESH, jj > ii)
        kf = jnp.where(kill, 1.0, 0.0)
        rr = lax.broadcasted_iota(jnp.int32, (_BJ, _BJ // 16), 0)
        ww = lax.broadcasted_iota(jnp.int32, (_BJ, _BJ // 16), 1)
        w16 = jnp.where(rr // 16 == ww,
                        lax.shift_left(jnp.int32(1), rr % 16),
                        0).astype(jnp.float32)
        packed = lax.dot(kf, w16, precision=lax.Precision.HIGHEST)
        m_ref[...] = jnp.reshape(packed.astype(jnp.int32), (2, 1, 128, 16))
        rs = jnp.sum(packed, axis=1, keepdims=True).astype(jnp.int32)
        any_ref[...] = jnp.where(jc == pi, rs, any_ref[...] + rs)


def _scan_sc(m3, anyrow, sorted_coords, kvec):
    info = plsc.get_sparse_core_info()
    mesh = plsc.VectorSubcoreMesh(core_axis_name="c", subcore_axis_name="s")

    @functools.partial(
        pl.kernel, mesh=mesh,
        compiler_params=pltpu.CompilerParams(needs_layout_passes=False, use_tc_tiling_on_sc=False),
        out_type=jax.ShapeDtypeStruct((_OUT_PAD * 4,), jnp.float32),
        scratch_types=[
            pltpu.VMEM((_NREG, _CHUNK, 16), jnp.int32),
            pltpu.VMEM((_NREG, _CHUNK, 16), jnp.int32),
            pltpu.VMEM((4, _PAD), jnp.float32),
            pltpu.VMEM((_PAD,), jnp.int32),
            pltpu.VMEM((_OUT_PAD,), jnp.int32),
            pltpu.VMEM((16,), jnp.int32),
            pltpu.VMEM((_NW,), jnp.int32),
            pltpu.VMEM((_OUT_PAD * 4,), jnp.float32),
            pltpu.SemaphoreType.DMA,
            pltpu.SemaphoreType.DMA,
        ],
    )
    def k(m_hbm, any_hbm, coords_hbm, k_hbm, out_hbm,
          mbufa, mbufb, coords_v, any_v, kept_v, kvec_v, sup_v, out_v,
          sema, semb):
        wid = lax.axis_index("s") * info.num_cores + lax.axis_index("c")

        @pl.when(wid == 0)
        def _():
            pltpu.sync_copy(k_hbm, kvec_v)
            pltpu.sync_copy(coords_hbm, coords_v)
            pltpu.sync_copy(any_hbm, any_v)
            iota16 = lax.iota(jnp.int32, 16)
            zi = jnp.zeros((16,), jnp.int32)
            zf = jnp.zeros((16,), jnp.float32)
            for t in range(_NREG):
                sup_v[pl.ds(t * 16, 16)] = zi
            kk = jnp.max(kvec_v[...])

            bufs = (mbufa, mbufb)
            sems = (sema, semb)

            def start_dma(c):
                t0 = c // 2
                return pltpu.async_copy(
                    m_hbm.at[c, pl.ds(t0, _NREG - t0)],
                    bufs[c % 2].at[pl.ds(t0, _NREG - t0)],
                    sems[c % 2])

            handles = {}
            for c in range(2):
                handles[c] = start_dma(c)

            p = jnp.int32(0)
            cnt = jnp.int32(0)

            for c in range(_NCHUNK):
                buf = bufs[c % 2]
                handles[c].wait()
                t_c = c // 2
                pend = jnp.minimum(kk, jnp.int32((c + 1) * _CHUNK))

                def cond(st):
                    pp, cc = st
                    return jnp.logical_and(pp < pend, cc < N_POST_NMS)

                def body(st, c=c, t_c=t_c, buf=buf):
                    pp, cc = st
                    sup_tc = sup_v[pl.ds(t_c * 16, 16)]
                    w = pp >> 4
                    wvalv = lax.gather(
                        sup_tc,
                        jnp.broadcast_to(w & 15, (16,))[:, None],
                        lax.GatherDimensionNumbers(
                            offset_dims=(), collapsed_slice_dims=(0,),
                            start_index_map=(0,)),
                        (1,),
                        mode=lax.GatherScatterMode.PROMISE_IN_BOUNDS)
                    bits = (wvalv >> iota16) & 1
                    posv = (w << 4) + iota16
                    avail = jnp.logical_and(
                        jnp.logical_and(bits == 0, posv >= pp), posv < kk)
                    ar = plsc.load_gather(any_v, [posv])
                    blocked = jnp.logical_and(avail, ar != 0)
                    pr = plsc.cumsum(jnp.where(avail, 1, 0))
                    fb = jnp.min(jnp.where(blocked, iota16, 16))
                    batch = jnp.logical_and(avail, iota16 < fb)
                    allowed = jnp.logical_and(
                        batch, pr <= jnp.int32(N_POST_NMS) - cc)
                    nk = jnp.max(jnp.where(allowed, pr, 0))
                    plsc.store_scatter(kept_v, [cc - 1 + pr], posv,
                                       mask=allowed)
                    cc2 = cc + nk

                    def keep_one(op):
                        pp, cc = op
                        pos = (w << 4) + fb
                        lr = pos - c * _CHUNK
                        for t in range(t_c, _NREG):
                            sup_v[pl.ds(t * 16, 16)] = jnp.bitwise_or(
                                sup_v[pl.ds(t * 16, 16)], buf[t, lr, :])
                        plsc.store_scatter(
                            kept_v, [jnp.broadcast_to(cc, (16,))],
                            jnp.broadcast_to(pos, (16,)),
                            mask=iota16 == 0)
                        return pos + 1, cc + 1

                    def skip(op):
                        pp, cc = op
                        return (w + 1) << 4, cc

                    return lax.cond(
                        jnp.logical_and(fb < 16, cc2 < N_POST_NMS),
                        keep_one, skip, (pp, cc2))

                p, cnt = lax.while_loop(cond, body, (p, cnt))
                if c + 2 < _NCHUNK:
                    handles[c + 2] = start_dma(c + 2)

            def glp(g, _):
                idx = kept_v[pl.ds(g * 16, 16)]
                valid = (g * 16 + iota16) < cnt
                base4 = (g * 16 + iota16) * 4
                for r in range(4):
                    vals = plsc.load_gather(
                        coords_v, [jnp.full((16,), r, jnp.int32), idx],
                        mask=valid)
                    plsc.store_scatter(out_v, [base4 + r], vals, mask=valid)
                return 0

            lax.fori_loop(0, (cnt + 15) >> 4, glp, 0)

            start = cnt * 4
            g0 = start >> 4
            plsc.store_scatter(out_v, [g0 * 16 + iota16], zf,
                               mask=(g0 * 16 + iota16) >= start)

            def zlp(g, _):
                out_v[pl.ds(g * 16, 16)] = zf
                return 0

            lax.fori_loop(g0 + 1, _OUT_PAD * 4 // 16, zlp, 0)
            pltpu.sync_copy(out_v, out_hbm)

    return k(m3, anyrow, sorted_coords, kvec)


def kernel(classification, regression, anchors, image_shape):
    sc = classification[0, :, classification.shape[-1] // 2]
    reg = jnp.reshape(regression[0], (-1, 4))
    anc = anchors[0]

    def p(v):
        return jnp.reshape(jnp.pad(v, (0, _PAD - N_BOXES)), (_ROWS, _LANES))

    vals = jnp.stack([
        p(sc),
        p(reg[:, 0]), p(reg[:, 1]), p(reg[:, 2]), p(reg[:, 3]),
        p(anc[:, 0]), p(anc[:, 1]), p(anc[:, 2]), p(anc[:, 3]),
    ])
    ims = jnp.reshape(image_shape.astype(jnp.float32), (1, 2))

    coords, scores, kc = pl.pallas_call(
        _decode_body,
        out_shape=(
            jax.ShapeDtypeStruct((4, _ROWS, _LANES), jnp.float32),
            jax.ShapeDtypeStruct((_ROWS, _LANES), jnp.float32),
            jax.ShapeDtypeStruct((1, 1), jnp.int32),
        ),
    )(vals, ims)

    srow = jnp.reshape(scores, (1, _PAD))
    ranks = pl.pallas_call(
        _rank_body,
        grid=(_PAD // _RB,),
        in_specs=[
            pl.BlockSpec((1, _RB), lambda i: (0, i)),
            pl.BlockSpec((1, _PAD), lambda i: (0, 0)),
        ],
        out_specs=pl.BlockSpec((1, _RB), lambda i: (0, i)),
        out_shape=jax.ShapeDtypeStruct((1, _PAD), jnp.int32),
    )(srow, srow)

    sorted_coords = _scatter_sc(jnp.reshape(coords, (4, _PAD)),
                                jnp.reshape(ranks, (_PAD,)))

    m = pl.pallas_call(
        _mask_body,
        grid=(_NB // 2, _NB + 1),
        in_specs=[
            pl.BlockSpec((_BI, 4), lambda i, j: (_tri(i, j)[0], 0)),
            pl.BlockSpec((4, _BJ), lambda i, j: (0, _tri(i, j)[1])),
        ],
        out_specs=(
            pl.BlockSpec((2, 1, _CHUNK, 16),
                         lambda i, j: (_tri(i, j)[0], _tri(i, j)[1], 0, 0)),
            pl.BlockSpec((_BI, 1), lambda i, j: (_tri(i, j)[0], 0)),
        ),
        out_shape=(
            jax.ShapeDtypeStruct((_NCHUNK, _NREG, _CHUNK, 16), jnp.int32),
            jax.ShapeDtypeStruct((_PAD, 1), jnp.int32),
        ),
    )(jnp.transpose(sorted_coords), sorted_coords)

    kvec = jnp.full((16,), 1, jnp.int32) * kc[0, 0]
    out_flat = _scan_sc(m[0], jnp.reshape(m[1], (_PAD,)), sorted_coords, kvec)
    return jnp.reshape(out_flat, (_OUT_PAD, 4))[None, :N_POST_NMS, :]
